# baseline (speedup 1.0000x reference)
import functools

import jax
import jax.numpy as jnp
from jax import lax
from jax.experimental import pallas as pl
from jax.experimental.pallas import tpu as pltpu
from jax.experimental.pallas import tpu_sc as plsc

K_TAPS = 4
DILATION = 2
PAD_P = 4


def kernel(x):
    B, C, T = x.shape
    R = B * C
    TP = T // 2
    x2 = x.reshape(R, TP, 2)

    info = plsc.get_sparse_core_info()
    num_workers = info.num_cores * info.num_subcores
    rows_per_w = R // num_workers
    assert rows_per_w * num_workers == R

    mesh = plsc.VectorSubcoreMesh(core_axis_name="c", subcore_axis_name="s")

    @functools.partial(
        pl.kernel,
        mesh=mesh,
        out_type=jax.ShapeDtypeStruct((B * K_TAPS * C, TP, 2), jnp.float32),
        scratch_types=[pltpu.VMEM((PAD_P + TP, 2), jnp.float32)],
        compiler_params=pltpu.CompilerParams(use_tc_tiling_on_sc=False),
    )
    def unfold(x_hbm, zeros_hbm, out_hbm, buf):
        cid = lax.axis_index("c")
        sid = lax.axis_index("s")
        wid = sid * info.num_cores + cid
        base = wid * rows_per_w
        pltpu.sync_copy(zeros_hbm, buf.at[pl.ds(0, PAD_P)])

        def body(i, carry):
            r = base + i
            b = r // C
            c = r - b * C
            pltpu.sync_copy(x_hbm.at[r], buf.at[pl.ds(PAD_P, TP)])
            for k in range(K_TAPS):
                sp = (K_TAPS - 1 - k) * DILATION // 2
                dst = (b * K_TAPS + k) * C + c
                pltpu.sync_copy(buf.at[pl.ds(PAD_P - sp, TP)], out_hbm.at[dst])
            return carry

        lax.fori_loop(0, rows_per_w, body, 0)

    zeros = jnp.zeros((PAD_P, 2), jnp.float32)
    out = unfold(x2, zeros)
    return out.reshape(B, K_TAPS * C, T)


# speedup vs baseline: 19.3708x; 19.3708x over previous
import functools

import jax
import jax.numpy as jnp
from jax import lax
from jax.experimental import pallas as pl
from jax.experimental.pallas import tpu as pltpu
from jax.experimental.pallas import tpu_sc as plsc

K_TAPS = 4
DILATION = 2
GUARD = 16
G = 3
NSLOT = 2


def kernel(x):
    B, C, T = x.shape
    R = B * C
    x1 = x.reshape(R * T)

    info = plsc.get_sparse_core_info()
    num_workers = info.num_cores * info.num_subcores
    rows_per_w = R // num_workers
    assert rows_per_w * num_workers == R
    n_chunks = rows_per_w // G
    assert n_chunks * G == rows_per_w
    assert C % rows_per_w == 0

    mesh = plsc.VectorSubcoreMesh(core_axis_name="c", subcore_axis_name="s")

    @functools.partial(
        pl.kernel,
        mesh=mesh,
        out_type=jax.ShapeDtypeStruct((B * K_TAPS * C * T,), jnp.float32),
        scratch_types=[
            pltpu.VMEM((GUARD + NSLOT * G * T,), jnp.float32),
            pltpu.VMEM((NSLOT * (K_TAPS - 1) * G * T,), jnp.float32),
            pltpu.SemaphoreType.DMA,
            pltpu.SemaphoreType.DMA,
            pltpu.SemaphoreType.DMA,
            pltpu.SemaphoreType.DMA,
            pltpu.SemaphoreType.DMA,
        ],
    )
    def unfold(x_hbm, out_hbm, inbuf, outbuf, in_sem0, in_sem1, out_sem0, out_sem1, pa_sem):
        cid = lax.axis_index("c")
        sid = lax.axis_index("s")
        wid = sid * info.num_cores + cid
        row0 = wid * rows_per_w
        b = row0 // C
        c0 = row0 - b * C
        in_sems = [in_sem0, in_sem1]
        out_sems = [out_sem0, out_sem1]

        def hbm_off(row_idx):
            return pl.multiple_of(row_idx * T, 8)

        pa_dst = (b * K_TAPS + (K_TAPS - 1)) * C + c0

        def phase_a():
            return pltpu.make_async_copy(
                x_hbm.at[pl.ds(hbm_off(row0), rows_per_w * T)],
                out_hbm.at[pl.ds(hbm_off(pa_dst), rows_per_w * T)],
                pa_sem,
            )

        phase_a().start()

        def load_chunk(c_idx, sl):
            return pltpu.make_async_copy(
                x_hbm.at[pl.ds(hbm_off(row0 + c_idx * G), G * T)],
                inbuf.at[pl.ds(GUARD + sl * G * T, G * T)],
                in_sems[sl],
            )

        def store_chunk(c_idx, sl, k):
            dst = (b * K_TAPS + k) * C + c0 + c_idx * G
            return pltpu.make_async_copy(
                outbuf.at[pl.ds((sl * (K_TAPS - 1) + k) * G * T, G * T)],
                out_hbm.at[pl.ds(hbm_off(dst), G * T)],
                out_sems[sl],
            )

        load_chunk(0, 0).start()
        load_chunk(1, 1).start()

        iota16 = lax.iota(jnp.int32, 16)
        zeros16 = jnp.zeros((16,), jnp.float32)

        def chunk_body(cc, carry):
            for sl in range(NSLOT):
                c_idx = cc * NSLOT + sl
                load_chunk(c_idx, sl).wait()

                @pl.when(cc >= 1)
                def _drain():
                    for k in range(K_TAPS - 1):
                        store_chunk(c_idx, sl, k).wait()

                in_base = GUARD + sl * G * T
                out_slot = sl * (K_TAPS - 1) * G * T

                for k in range(K_TAPS - 1):
                    s = (K_TAPS - 1 - k) * DILATION
                    for g in range(G):
                        v = inbuf[pl.ds(in_base + g * T - s, 16)]
                        fixed = jnp.where(iota16 < s, 0.0, v)
                        outbuf[pl.ds(out_slot + (k * G + g) * T, 16)] = fixed

                def jbody(j, jcarry):
                    t16 = j * 16
                    for k in range(K_TAPS - 1):
                        s = (K_TAPS - 1 - k) * DILATION
                        for g in range(G):
                            v = inbuf[pl.ds(in_base + g * T + t16 - s, 16)]
                            outbuf[pl.ds(pl.multiple_of(out_slot + (k * G + g) * T + t16, 16), 16)] = v
                    return jcarry

                lax.fori_loop(1, T // 16, jbody, 0)

                for k in range(K_TAPS - 1):
                    store_chunk(c_idx, sl, k).start()

                @pl.when(cc < n_chunks // NSLOT - 1)
                def _next_load():
                    load_chunk(c_idx + NSLOT, sl).start()

            return carry

        lax.fori_loop(0, n_chunks // NSLOT, chunk_body, 0)

        for sl in range(NSLOT):
            for k in range(K_TAPS - 1):
                store_chunk(n_chunks - NSLOT + sl, sl, k).wait()
        phase_a().wait()

    out = unfold(x1)
    return out.reshape(B, K_TAPS * C, T)


# speedup vs baseline: 19.3819x; 1.0006x over previous
import functools

import jax
import jax.numpy as jnp
from jax import lax
from jax.experimental import pallas as pl
from jax.experimental.pallas import tpu as pltpu
from jax.experimental.pallas import tpu_sc as plsc

K_TAPS = 4
DILATION = 2
GUARD = 16
G = 3
NSLOT = 2


def kernel(x):
    B, C, T = x.shape
    R = B * C
    x1 = x.reshape(R * T)

    info = plsc.get_sparse_core_info()
    num_workers = info.num_cores * info.num_subcores
    rows_per_w = R // num_workers
    assert rows_per_w * num_workers == R
    n_chunks = rows_per_w // G
    assert n_chunks * G == rows_per_w
    assert C % rows_per_w == 0

    mesh = plsc.VectorSubcoreMesh(core_axis_name="c", subcore_axis_name="s")

    @functools.partial(
        pl.kernel,
        mesh=mesh,
        out_type=jax.ShapeDtypeStruct((B * K_TAPS * C * T,), jnp.float32),
        scratch_types=[
            pltpu.VMEM((GUARD + NSLOT * G * T,), jnp.float32),
            pltpu.VMEM((NSLOT * (K_TAPS - 1) * G * T,), jnp.float32),
            pltpu.SemaphoreType.DMA,
            pltpu.SemaphoreType.DMA,
            pltpu.SemaphoreType.DMA,
            pltpu.SemaphoreType.DMA,
            pltpu.SemaphoreType.DMA,
        ],
    )
    def unfold(x_hbm, out_hbm, inbuf, outbuf, in_sem0, in_sem1, out_sem0, out_sem1, pa_sem):
        cid = lax.axis_index("c")
        sid = lax.axis_index("s")
        wid = sid * info.num_cores + cid
        row0 = wid * rows_per_w
        b = row0 // C
        c0 = row0 - b * C
        in_sems = [in_sem0, in_sem1]
        out_sems = [out_sem0, out_sem1]

        def hbm_off(row_idx):
            return pl.multiple_of(row_idx * T, 8)

        pa_dst = (b * K_TAPS + (K_TAPS - 1)) * C + c0

        def phase_a():
            return pltpu.make_async_copy(
                x_hbm.at[pl.ds(hbm_off(row0), rows_per_w * T)],
                out_hbm.at[pl.ds(hbm_off(pa_dst), rows_per_w * T)],
                pa_sem,
            )

        phase_a().start()

        def load_chunk(c_idx, sl):
            return pltpu.make_async_copy(
                x_hbm.at[pl.ds(hbm_off(row0 + c_idx * G), G * T)],
                inbuf.at[pl.ds(GUARD + sl * G * T, G * T)],
                in_sems[sl],
            )

        def store_chunk(c_idx, sl, k):
            dst = (b * K_TAPS + k) * C + c0 + c_idx * G
            return pltpu.make_async_copy(
                outbuf.at[pl.ds((sl * (K_TAPS - 1) + k) * G * T, G * T)],
                out_hbm.at[pl.ds(hbm_off(dst), G * T)],
                out_sems[sl],
            )

        load_chunk(0, 0).start()
        load_chunk(1, 1).start()

        iota16 = lax.iota(jnp.int32, 16)
        zeros16 = jnp.zeros((16,), jnp.float32)

        def chunk_body(cc, carry):
            for sl in range(NSLOT):
                c_idx = cc * NSLOT + sl
                load_chunk(c_idx, sl).wait()

                @pl.when(cc >= 1)
                def _drain():
                    for k in range(K_TAPS - 1):
                        store_chunk(c_idx, sl, k).wait()

                in_base = GUARD + sl * G * T
                out_slot = sl * (K_TAPS - 1) * G * T

                for k in range(K_TAPS - 1):
                    s = (K_TAPS - 1 - k) * DILATION
                    for g in range(G):
                        v = inbuf[pl.ds(in_base + g * T - s, 16)]
                        fixed = jnp.where(iota16 < s, 0.0, v)
                        outbuf[pl.ds(out_slot + (k * G + g) * T, 16)] = fixed

                @plsc.parallel_loop(1, T // 16, unroll=4)
                def jbody(j):
                    t16 = j * 16
                    for k in range(K_TAPS - 1):
                        s = (K_TAPS - 1 - k) * DILATION
                        for g in range(G):
                            v = inbuf[pl.ds(in_base + g * T + t16 - s, 16)]
                            outbuf[pl.ds(pl.multiple_of(out_slot + (k * G + g) * T + t16, 16), 16)] = v

                for k in range(K_TAPS - 1):
                    store_chunk(c_idx, sl, k).start()

                @pl.when(cc < n_chunks // NSLOT - 1)
                def _next_load():
                    load_chunk(c_idx + NSLOT, sl).start()

            return carry

        lax.fori_loop(0, n_chunks // NSLOT, chunk_body, 0)

        for sl in range(NSLOT):
            for k in range(K_TAPS - 1):
                store_chunk(n_chunks - NSLOT + sl, sl, k).wait()
        phase_a().wait()

    out = unfold(x1)
    return out.reshape(B, K_TAPS * C, T)


# speedup vs baseline: 423.8560x; 21.8687x over previous
import jax
import jax.numpy as jnp
from jax.experimental import pallas as pl

K_TAPS = 4
DILATION = 2
BC = 128


def kernel(x):
    B, C, T = x.shape

    def body(x_ref, o_ref):
        xv = x_ref[0]
        for k in range(K_TAPS):
            s = (K_TAPS - 1 - k) * DILATION
            if s == 0:
                o_ref[0, k] = xv
            else:
                o_ref[0, k] = jnp.concatenate(
                    [jnp.zeros((BC, s), jnp.float32), xv[:, : T - s]], axis=1
                )

    out4 = pl.pallas_call(
        body,
        grid=(B, C // BC),
        in_specs=[pl.BlockSpec((1, BC, T), lambda b, c: (b, c, 0))],
        out_specs=pl.BlockSpec((1, K_TAPS, BC, T), lambda b, c: (b, 0, c, 0)),
        out_shape=jax.ShapeDtypeStruct((B, K_TAPS, C, T), jnp.float32),
    )(x)
    return out4.reshape(B, K_TAPS * C, T)
